# Initial kernel scaffold; baseline (speedup 1.0000x reference)
import jax
import jax.numpy as jnp
from jax import lax
from jax.experimental import pallas as pl
from jax.experimental.pallas import tpu as pltpu
from jax.experimental.pallas import tpu_sc as plsc

N_NODES = 10000
N_EDGES = 320000
D = 128

NC = 2
NS = 16
NW = NC * NS
CHUNK = 128
N_PAD = 10240
ROWS_PER_TILE = N_PAD // NS

_N_CHUNKS = -(-N_EDGES // (NW * CHUNK))
E_PAD = NW * CHUNK * _N_CHUNKS
EDGES_PER_TILE = _N_CHUNKS * CHUNK


def _proj_body(x_ref, w_ref, asrc_ref, adst_ref, h_ref, es_ref, ed_ref):
    x = x_ref[...]
    w = w_ref[...]
    h = lax.dot_general(x, w, (((1,), (1,)), ((), ())),
                        preferred_element_type=jnp.float32)
    h_ref[...] = h
    es_ref[...] = jnp.sum(h * asrc_ref[...], axis=1, keepdims=True)
    ed_ref[...] = jnp.sum(h * adst_ref[...], axis=1, keepdims=True)


def _project(X, W, asrc, adst):
    blk = 2000
    grid = N_NODES // blk
    return pl.pallas_call(
        _proj_body,
        grid=(grid,),
        in_specs=[
            pl.BlockSpec((blk, D), lambda g: (g, 0)),
            pl.BlockSpec((D, D), lambda g: (0, 0)),
            pl.BlockSpec((1, D), lambda g: (0, 0)),
            pl.BlockSpec((1, D), lambda g: (0, 0)),
        ],
        out_specs=[
            pl.BlockSpec((blk, D), lambda g: (g, 0)),
            pl.BlockSpec((blk, 1), lambda g: (g, 0)),
            pl.BlockSpec((blk, 1), lambda g: (g, 0)),
        ],
        out_shape=[
            jax.ShapeDtypeStruct((N_NODES, D), jnp.float32),
            jax.ShapeDtypeStruct((N_NODES, 1), jnp.float32),
            jax.ShapeDtypeStruct((N_NODES, 1), jnp.float32),
        ],
    )(X, W, asrc, adst)


def _sc_body(i_hbm, j_hbm, es_hbm, ed_hbm, h_hbm,
             acc_hbm, den_hbm,
             es_v, ed_v, ii_v, jj_v, xe_v, rows_v,
             zrow_v,
             acc_sh, den_sh,
             sem):
    c = lax.axis_index("c")
    s = lax.axis_index("s")
    wid = c * NS + s

    zero16 = jnp.zeros((16,), jnp.float32)

    def zrow_init(k, _):
        zrow_v[k, ...] = jnp.zeros((CHUNK // 16, 16), jnp.float32) * 0.0 \
            if False else zero16
        return 0

    def zrow_init2(k, _):
        zrow_v[pl.ds(0, CHUNK), :][...]
        return 0

    def zinit(k, _):
        zflat = zrow_v.reshape(CHUNK * D)
        zflat[pl.ds(k * 16, 16)] = zero16
        return 0
    lax.fori_loop(0, (CHUNK * D) // 16, zinit, 0)

    row0 = s * ROWS_PER_TILE
    for b in range(ROWS_PER_TILE // CHUNK):
        pltpu.sync_copy(zrow_v, acc_sh.at[pl.ds(row0 + b * CHUNK, CHUNK)])
    zflat = zrow_v.reshape(CHUNK * D)
    for b in range(ROWS_PER_TILE // CHUNK):
        pltpu.sync_copy(zflat.at[pl.ds(0, CHUNK)],
                        den_sh.at[pl.ds(row0 + b * CHUNK, CHUNK)])

    pltpu.sync_copy(es_hbm, es_v)
    pltpu.sync_copy(ed_hbm, ed_v)

    plsc.subcore_barrier()

    edge0 = wid * EDGES_PER_TILE

    def chunk_body(cidx, _):
        base = edge0 + cidx * CHUNK
        pltpu.sync_copy(i_hbm.at[pl.ds(base, CHUNK)], ii_v)
        pltpu.sync_copy(j_hbm.at[pl.ds(base, CHUNK)], jj_v)

        for m in range(CHUNK // 16):
            iv = ii_v[pl.ds(m * 16, 16)]
            jv = jj_v[pl.ds(m * 16, 16)]
            e = plsc.load_gather(es_v, [iv]) + plsc.load_gather(ed_v, [jv])
            e = jnp.where(e > 0, e, 0.2 * e)
            e = jnp.minimum(jnp.maximum(e, -10.0), 10.0)
            xe_v[pl.ds(m * 16, 16)] = jnp.exp(e)

        pltpu.sync_copy(xe_v, den_sh.at[ii_v], add=True)

        pltpu.async_copy(h_hbm.at[jj_v], rows_v, sem).wait()

        def scale_body(k, _):
            kv = jnp.zeros((16,), jnp.int32) + k
            av = plsc.load_gather(xe_v, [kv])
            for mm in range(D // 16):
                sl = pl.ds(mm * 16, 16)
                rows_v[k, sl] = rows_v[k, sl] * av
            return 0
        lax.fori_loop(0, CHUNK, scale_body, 0)

        pltpu.sync_copy(rows_v, acc_sh.at[ii_v], add=True)
        return 0

    lax.fori_loop(0, _N_CHUNKS, chunk_body, 0)

    plsc.subcore_barrier()

    pltpu.sync_copy(acc_sh.at[pl.ds(row0, ROWS_PER_TILE)],
                    acc_hbm.at[c, pl.ds(row0, ROWS_PER_TILE), :])
    pltpu.sync_copy(den_sh.at[pl.ds(row0, ROWS_PER_TILE)],
                    den_hbm.at[c, pl.ds(row0, ROWS_PER_TILE)])


def _sc_edge(i_pad, j_pad, es_pad, ed_pad, h):
    mesh = plsc.VectorSubcoreMesh(core_axis_name="c", subcore_axis_name="s")
    f = pl.kernel(
        _sc_body,
        out_type=[
            jax.ShapeDtypeStruct((NC, N_PAD, D), jnp.float32),
            jax.ShapeDtypeStruct((NC, N_PAD), jnp.float32),
        ],
        mesh=mesh,
        scratch_types=[
            pltpu.VMEM((N_PAD,), jnp.float32),
            pltpu.VMEM((N_PAD,), jnp.float32),
            pltpu.VMEM((CHUNK,), jnp.int32),
            pltpu.VMEM((CHUNK,), jnp.int32),
            pltpu.VMEM((CHUNK,), jnp.float32),
            pltpu.VMEM((CHUNK, D), jnp.float32),
            pltpu.VMEM((CHUNK, D), jnp.float32),
            pltpu.VMEM_SHARED((N_PAD, D), jnp.float32),
            pltpu.VMEM_SHARED((N_PAD,), jnp.float32),
            pltpu.SemaphoreType.DMA,
        ],
    )
    return f(i_pad, j_pad, es_pad, ed_pad, h)


def _fin_body(acc_ref, den_ref, out_ref):
    a = acc_ref[0] + acc_ref[1]
    d = den_ref[0] + den_ref[1] + 1e-12
    out_ref[...] = a / d


def _finalize(acc, den3):
    blk = 2000
    grid = N_NODES // blk
    return pl.pallas_call(
        _fin_body,
        grid=(grid,),
        in_specs=[
            pl.BlockSpec((NC, blk, D), lambda g: (0, g, 0)),
            pl.BlockSpec((NC, blk, 1), lambda g: (0, g, 0)),
        ],
        out_specs=pl.BlockSpec((blk, D), lambda g: (g, 0)),
        out_shape=jax.ShapeDtypeStruct((N_NODES, D), jnp.float32),
    )(acc, den3)


@jax.jit
def kernel(X, edge_index, W, att_src, att_dst):
    i = edge_index[0].astype(jnp.int32)
    j = edge_index[1].astype(jnp.int32)
    pad = E_PAD - N_EDGES
    i_pad = jnp.concatenate([i, jnp.full((pad,), N_NODES, jnp.int32)])
    j_pad = jnp.concatenate([j, jnp.zeros((pad,), jnp.int32)])

    h, es, ed = _project(X, W, att_src.reshape(1, D), att_dst.reshape(1, D))
    es_pad = jnp.pad(es.reshape(-1), (0, N_PAD - N_NODES))
    ed_pad = jnp.pad(ed.reshape(-1), (0, N_PAD - N_NODES))

    acc, den = _sc_edge(i_pad, j_pad, es_pad, ed_pad, h)
    return _finalize(acc, den.reshape(NC, N_PAD, 1))


# speedup vs baseline: 14.5401x; 14.5401x over previous
import jax
import jax.numpy as jnp
from jax import lax
from jax.experimental import pallas as pl
from jax.experimental.pallas import tpu as pltpu
from jax.experimental.pallas import tpu_sc as plsc

N_NODES = 10000
N_EDGES = 320000
D = 128

NC = 2
NS = 16
NW = NC * NS
CHUNK = 128
N_PAD = 10240
ROWS_PER_TILE = N_PAD // NS

_N_CHUNKS = -(-N_EDGES // (NW * CHUNK))
E_PAD = NW * CHUNK * _N_CHUNKS
EDGES_PER_TILE = _N_CHUNKS * CHUNK


def _proj_body(x_ref, w_ref, asrc_ref, adst_ref, h_ref, es_ref, ed_ref):
    x = x_ref[...]
    w = w_ref[...]
    h = lax.dot_general(x, w, (((1,), (1,)), ((), ())),
                        preferred_element_type=jnp.float32)
    h_ref[...] = h
    es_ref[...] = jnp.sum(h * asrc_ref[...], axis=1, keepdims=True)
    ed_ref[...] = jnp.sum(h * adst_ref[...], axis=1, keepdims=True)


def _project(X, W, asrc, adst):
    blk = 2000
    grid = N_NODES // blk
    return pl.pallas_call(
        _proj_body,
        grid=(grid,),
        in_specs=[
            pl.BlockSpec((blk, D), lambda g: (g, 0)),
            pl.BlockSpec((D, D), lambda g: (0, 0)),
            pl.BlockSpec((1, D), lambda g: (0, 0)),
            pl.BlockSpec((1, D), lambda g: (0, 0)),
        ],
        out_specs=[
            pl.BlockSpec((blk, D), lambda g: (g, 0)),
            pl.BlockSpec((blk, 1), lambda g: (g, 0)),
            pl.BlockSpec((blk, 1), lambda g: (g, 0)),
        ],
        out_shape=[
            jax.ShapeDtypeStruct((N_NODES, D), jnp.float32),
            jax.ShapeDtypeStruct((N_NODES, 1), jnp.float32),
            jax.ShapeDtypeStruct((N_NODES, 1), jnp.float32),
        ],
    )(X, W, asrc, adst)


def _sc_body(i_hbm, j_hbm, es_hbm, ed_hbm, h_hbm,
             acc_hbm, den_hbm,
             es_v, ed_v, ii_v, jj_v, xe_v, rows_v,
             acc_sh, den_sh,
             sem):
    c = lax.axis_index("c")
    s = lax.axis_index("s")
    wid = c * NS + s

    zero16 = jnp.zeros((16,), jnp.float32)

    def zinit(k, _):
        for m in range(D // 16):
            rows_v[k, pl.ds(m * 16, 16)] = zero16
        return 0
    lax.fori_loop(0, CHUNK, zinit, 0)
    for m in range(CHUNK // 16):
        xe_v[pl.ds(m * 16, 16)] = zero16

    row0 = s * ROWS_PER_TILE
    for b in range(ROWS_PER_TILE // CHUNK):
        pltpu.sync_copy(rows_v, acc_sh.at[pl.ds(row0 + b * CHUNK, CHUNK)])
        pltpu.sync_copy(xe_v, den_sh.at[pl.ds(row0 + b * CHUNK, CHUNK)])

    pltpu.sync_copy(es_hbm, es_v)
    pltpu.sync_copy(ed_hbm, ed_v)

    plsc.subcore_barrier()

    edge0 = wid * EDGES_PER_TILE

    def chunk_body(cidx, _):
        base = edge0 + cidx * CHUNK
        pltpu.sync_copy(i_hbm.at[pl.ds(base, CHUNK)], ii_v)
        pltpu.sync_copy(j_hbm.at[pl.ds(base, CHUNK)], jj_v)

        for m in range(CHUNK // 16):
            iv = ii_v[pl.ds(m * 16, 16)]
            jv = jj_v[pl.ds(m * 16, 16)]
            e = plsc.load_gather(es_v, [iv]) + plsc.load_gather(ed_v, [jv])
            e = jnp.where(e > 0, e, 0.2 * e)
            e = jnp.minimum(jnp.maximum(e, -10.0), 10.0)
            xe_v[pl.ds(m * 16, 16)] = jnp.exp(e)

        pltpu.sync_copy(xe_v, den_sh.at[ii_v], add=True)

        pltpu.async_copy(h_hbm.at[jj_v], rows_v, sem).wait()

        def scale_body(k, _):
            kv = jnp.zeros((16,), jnp.int32) + k
            av = plsc.load_gather(xe_v, [kv])
            for mm in range(D // 16):
                sl = pl.ds(mm * 16, 16)
                rows_v[k, sl] = rows_v[k, sl] * av
            return 0
        lax.fori_loop(0, CHUNK, scale_body, 0)

        pltpu.sync_copy(rows_v, acc_sh.at[ii_v], add=True)
        return 0

    lax.fori_loop(0, _N_CHUNKS, chunk_body, 0)

    plsc.subcore_barrier()

    pltpu.sync_copy(acc_sh.at[pl.ds(row0, ROWS_PER_TILE)],
                    acc_hbm.at[c, pl.ds(row0, ROWS_PER_TILE), :])
    pltpu.sync_copy(den_sh.at[pl.ds(row0, ROWS_PER_TILE)],
                    den_hbm.at[c, pl.ds(row0, ROWS_PER_TILE)])


def _sc_edge(i_pad, j_pad, es_pad, ed_pad, h):
    mesh = plsc.VectorSubcoreMesh(core_axis_name="c", subcore_axis_name="s")
    f = pl.kernel(
        _sc_body,
        out_type=[
            jax.ShapeDtypeStruct((NC, N_PAD, D), jnp.float32),
            jax.ShapeDtypeStruct((NC, N_PAD), jnp.float32),
        ],
        mesh=mesh,
        compiler_params=pltpu.CompilerParams(needs_layout_passes=False),
        scratch_types=[
            pltpu.VMEM((N_PAD,), jnp.float32),
            pltpu.VMEM((N_PAD,), jnp.float32),
            pltpu.VMEM((CHUNK,), jnp.int32),
            pltpu.VMEM((CHUNK,), jnp.int32),
            pltpu.VMEM((CHUNK,), jnp.float32),
            pltpu.VMEM((CHUNK, D), jnp.float32),
            pltpu.VMEM_SHARED((N_PAD, D), jnp.float32),
            pltpu.VMEM_SHARED((N_PAD,), jnp.float32),
            pltpu.SemaphoreType.DMA,
        ],
    )
    return f(i_pad, j_pad, es_pad, ed_pad, h)


def _fin_body(acc_ref, den_ref, out_ref):
    a = acc_ref[0] + acc_ref[1]
    d = den_ref[0] + den_ref[1] + 1e-12
    out_ref[...] = a / d


def _finalize(acc, den3):
    blk = 2000
    grid = N_NODES // blk
    return pl.pallas_call(
        _fin_body,
        grid=(grid,),
        in_specs=[
            pl.BlockSpec((NC, blk, D), lambda g: (0, g, 0)),
            pl.BlockSpec((NC, blk, 1), lambda g: (0, g, 0)),
        ],
        out_specs=pl.BlockSpec((blk, D), lambda g: (g, 0)),
        out_shape=jax.ShapeDtypeStruct((N_NODES, D), jnp.float32),
    )(acc, den3)


@jax.jit
def kernel(X, edge_index, W, att_src, att_dst):
    i = edge_index[0].astype(jnp.int32)
    j = edge_index[1].astype(jnp.int32)
    pad = E_PAD - N_EDGES
    i_pad = jnp.concatenate([i, jnp.full((pad,), N_NODES, jnp.int32)])
    j_pad = jnp.concatenate([j, jnp.zeros((pad,), jnp.int32)])

    h, es, ed = _project(X, W, att_src.reshape(1, D), att_dst.reshape(1, D))
    es_pad = jnp.pad(es.reshape(-1), (0, N_PAD - N_NODES))
    ed_pad = jnp.pad(ed.reshape(-1), (0, N_PAD - N_NODES))

    acc, den = _sc_edge(i_pad, j_pad, es_pad, ed_pad, h)
    return _finalize(acc, den.reshape(NC, N_PAD, 1))


# speedup vs baseline: 15.7381x; 1.0824x over previous
import jax
import jax.numpy as jnp
from jax import lax
from jax.experimental import pallas as pl
from jax.experimental.pallas import tpu as pltpu
from jax.experimental.pallas import tpu_sc as plsc

N_NODES = 10000
N_EDGES = 320000
D = 128

NC = 2
NS = 16
NW = NC * NS
CHUNK = 128
BLK = 16
NBLK = 5
N_CHUNKS = BLK * NBLK
N_PAD = 10240
ROWS_PER_TILE = N_PAD // NS

EDGES_PER_TILE = N_CHUNKS * CHUNK
E_PAD = NW * EDGES_PER_TILE
E_ROWS = E_PAD // CHUNK


def _proj_body(x_ref, w_ref, asrc_ref, adst_ref, h_ref, es_ref, ed_ref):
    x = x_ref[...]
    w = w_ref[...]
    h = lax.dot_general(x, w, (((1,), (1,)), ((), ())),
                        preferred_element_type=jnp.float32)
    h_ref[...] = h
    es_ref[...] = jnp.sum(h * asrc_ref[...], axis=1, keepdims=True)
    ed_ref[...] = jnp.sum(h * adst_ref[...], axis=1, keepdims=True)


def _project(X, W, asrc, adst):
    blk = 2000
    grid = N_NODES // blk
    return pl.pallas_call(
        _proj_body,
        grid=(grid,),
        in_specs=[
            pl.BlockSpec((blk, D), lambda g: (g, 0)),
            pl.BlockSpec((D, D), lambda g: (0, 0)),
            pl.BlockSpec((1, D), lambda g: (0, 0)),
            pl.BlockSpec((1, D), lambda g: (0, 0)),
        ],
        out_specs=[
            pl.BlockSpec((blk, D), lambda g: (g, 0)),
            pl.BlockSpec((blk, 1), lambda g: (g, 0)),
            pl.BlockSpec((blk, 1), lambda g: (g, 0)),
        ],
        out_shape=[
            jax.ShapeDtypeStruct((N_NODES, D), jnp.float32),
            jax.ShapeDtypeStruct((N_NODES, 1), jnp.float32),
            jax.ShapeDtypeStruct((N_NODES, 1), jnp.float32),
        ],
    )(X, W, asrc, adst)


def _xe_body(i2, j2, es_hbm, ed_hbm,
             xe2, den_hbm,
             es_v, ed_v, iiA, iiB, iiC, jjA, jjB, xeA, xeB, xeC, zv, iv,
             den_sh,
             bsem0, bsem1, xsem0, xsem1, xsem2, d0, d1, d2, d3):
    c = lax.axis_index("c")
    s = lax.axis_index("s")
    wid = c * NS + s
    row0 = s * ROWS_PER_TILE
    ii = (iiA, iiB, iiC)
    jj = (jjA, jjB)
    xev = (xeA, xeB, xeC)
    bsem = (bsem0, bsem1)
    xsem = (xsem0, xsem1, xsem2)
    dsem = (d0, d1, d2, d3)
    rbase = wid * N_CHUNKS

    pltpu.async_copy(i2.at[pl.ds(rbase, BLK)], iiA, bsem[0])
    pltpu.async_copy(j2.at[pl.ds(rbase, BLK)], jjA, bsem[0])

    zero16 = jnp.zeros((16,), jnp.float32)
    for m in range(8):
        zv[pl.ds(m * 16, 16)] = zero16
        iv[pl.ds(m * 16, 16)] = lax.iota(jnp.int32, 16) + (row0 + m * 16)
    for b5 in range(ROWS_PER_TILE // 128):
        pltpu.sync_copy(zv, den_sh.at[pl.ds(row0 + b5 * 128, 128)])

    pltpu.sync_copy(es_hbm, es_v)
    pltpu.sync_copy(ed_hbm, ed_v)

    plsc.subcore_barrier()

    for q in range(4):
        pltpu.async_copy(zv, den_sh.at[iv], dsem[q], add=True)

    def compute_chunk(buf2, buf3, r):
        for m in range(CHUNK // 16):
            sl = pl.ds(m * 16, 16)
            e = (plsc.load_gather(es_v, [ii[buf3][r, sl]])
                 + plsc.load_gather(ed_v, [jj[buf2][r, sl]]))
            e = jnp.where(e > 0, e, 0.2 * e)
            e = jnp.minimum(jnp.maximum(e, -10.0), 10.0)
            xev[buf3][r, sl] = jnp.exp(e)

    for blk in range(NBLK):
        buf2 = blk % 2
        buf3 = blk % 3
        if blk + 1 < NBLK:
            nb = rbase + (blk + 1) * BLK
            pltpu.async_copy(i2.at[pl.ds(nb, BLK)], ii[(blk + 1) % 3],
                             bsem[1 - buf2])
            pltpu.async_copy(j2.at[pl.ds(nb, BLK)], jj[1 - buf2],
                             bsem[1 - buf2])
        bb = rbase + blk * BLK
        pltpu.make_async_copy(i2.at[pl.ds(bb, BLK)], ii[buf3],
                              bsem[buf2]).wait()
        pltpu.make_async_copy(j2.at[pl.ds(bb, BLK)], jj[buf2],
                              bsem[buf2]).wait()
        if blk >= 3:
            pb = rbase + (blk - 3) * BLK
            pltpu.make_async_copy(xev[buf3], xe2.at[pl.ds(pb, BLK)],
                                  xsem[buf3]).wait()

        def quad(t, _):
            for u in range(4):
                r = t * 4 + u
                pltpu.make_async_copy(zv, den_sh.at[iv], dsem[u]).wait()
                compute_chunk(buf2, buf3, r)
                pltpu.async_copy(xev[buf3].at[r], den_sh.at[ii[buf3].at[r]],
                                 dsem[u], add=True)
            return 0
        lax.fori_loop(0, BLK // 4, quad, 0)

        pltpu.async_copy(xev[buf3], xe2.at[pl.ds(bb, BLK)], xsem[buf3])

    for q in range(4):
        pltpu.make_async_copy(zv, den_sh.at[iv], dsem[q]).wait()
    for blk in (NBLK - 3, NBLK - 2, NBLK - 1):
        buf3 = blk % 3
        bb = rbase + blk * BLK
        pltpu.make_async_copy(xev[buf3], xe2.at[pl.ds(bb, BLK)],
                              xsem[buf3]).wait()

    plsc.subcore_barrier()
    pltpu.sync_copy(den_sh.at[pl.ds(row0, ROWS_PER_TILE)],
                    den_hbm.at[c, pl.ds(row0, ROWS_PER_TILE)])


def _sc_scores(i2, j2, es_pad, ed_pad):
    mesh = plsc.VectorSubcoreMesh(core_axis_name="c", subcore_axis_name="s")
    f = pl.kernel(
        _xe_body,
        out_type=[
            jax.ShapeDtypeStruct((E_ROWS, CHUNK), jnp.float32),
            jax.ShapeDtypeStruct((NC, N_PAD), jnp.float32),
        ],
        mesh=mesh,
        compiler_params=pltpu.CompilerParams(needs_layout_passes=False),
        scratch_types=[
            pltpu.VMEM((N_PAD,), jnp.float32),
            pltpu.VMEM((N_PAD,), jnp.float32),
            pltpu.VMEM((BLK, CHUNK), jnp.int32),
            pltpu.VMEM((BLK, CHUNK), jnp.int32),
            pltpu.VMEM((BLK, CHUNK), jnp.int32),
            pltpu.VMEM((BLK, CHUNK), jnp.int32),
            pltpu.VMEM((BLK, CHUNK), jnp.int32),
            pltpu.VMEM((BLK, CHUNK), jnp.float32),
            pltpu.VMEM((BLK, CHUNK), jnp.float32),
            pltpu.VMEM((BLK, CHUNK), jnp.float32),
            pltpu.VMEM((CHUNK,), jnp.float32),
            pltpu.VMEM((CHUNK,), jnp.int32),
            pltpu.VMEM_SHARED((N_PAD,), jnp.float32),
            pltpu.SemaphoreType.DMA,
            pltpu.SemaphoreType.DMA,
            pltpu.SemaphoreType.DMA,
            pltpu.SemaphoreType.DMA,
            pltpu.SemaphoreType.DMA,
            pltpu.SemaphoreType.DMA,
            pltpu.SemaphoreType.DMA,
            pltpu.SemaphoreType.DMA,
            pltpu.SemaphoreType.DMA,
        ],
    )
    return f(i2, j2, es_pad, ed_pad)


def _rows_body(i2, j2, xe2, h_hbm,
               acc_hbm,
               iiA, iiB, iiC, jjA, jjB, xeA, xeB, rows0, rows1, xr,
               acc_sh,
               bsem0, bsem1, gsem0, gsem1, ssem0, ssem1):
    c = lax.axis_index("c")
    s = lax.axis_index("s")
    wid = c * NS + s
    row0 = s * ROWS_PER_TILE
    ii = (iiA, iiB, iiC)
    jj = (jjA, jjB)
    xev = (xeA, xeB)
    rows = (rows0, rows1)
    bsem = (bsem0, bsem1)
    gsem = (gsem0, gsem1)
    ssem = (ssem0, ssem1)
    rbase = wid * N_CHUNKS

    pltpu.async_copy(i2.at[pl.ds(rbase, BLK)], iiA, bsem[0])
    pltpu.async_copy(j2.at[pl.ds(rbase, BLK)], jjA, bsem[0])
    pltpu.async_copy(xe2.at[pl.ds(rbase, BLK)], xeA, bsem[0])

    zero16 = jnp.zeros((16,), jnp.float32)

    def zinit(k, _):
        for m in range(D // 16):
            rows0[k, pl.ds(m * 16, 16)] = zero16
        return 0
    lax.fori_loop(0, CHUNK, zinit, 0)
    for b5 in range(ROWS_PER_TILE // CHUNK):
        pltpu.sync_copy(rows0, acc_sh.at[pl.ds(row0 + b5 * CHUNK, CHUNK)])

    plsc.subcore_barrier()

    zero16i = jnp.zeros((16,), jnp.int32)

    def scale(rowbuf, xebuf, r):
        for m in range(CHUNK // 16):
            sl = pl.ds(m * 16, 16)
            xr[sl] = xev[xebuf][r, sl]

        def scale_body(k4, _):
            k = k4 * 4
            for u in range(4):
                av = plsc.load_gather(xr, [zero16i + (k + u)])
                for mm in range(D // 16):
                    sl = pl.ds(mm * 16, 16)
                    rows[rowbuf][k + u, sl] = rows[rowbuf][k + u, sl] * av
            return 0
        lax.fori_loop(0, CHUNK // 4, scale_body, 0)

    def g_start(buf, buf2, r):
        pltpu.async_copy(h_hbm.at[jj[buf2].at[r]], rows[buf], gsem[buf])

    def g_wait(buf, buf2, r):
        pltpu.make_async_copy(h_hbm.at[jj[buf2].at[r]], rows[buf],
                              gsem[buf]).wait()

    def s_start(buf, buf3, r):
        pltpu.async_copy(rows[buf], acc_sh.at[ii[buf3].at[r]], ssem[buf],
                         add=True)

    def s_wait(buf, buf3, r):
        pltpu.make_async_copy(rows[buf], acc_sh.at[ii[buf3].at[r]],
                              ssem[buf]).wait()

    for blk in range(NBLK):
        buf2 = blk % 2
        buf3 = blk % 3
        if blk + 1 < NBLK:
            nb = rbase + (blk + 1) * BLK
            pltpu.async_copy(i2.at[pl.ds(nb, BLK)], ii[(blk + 1) % 3],
                             bsem[1 - buf2])
            pltpu.async_copy(j2.at[pl.ds(nb, BLK)], jj[1 - buf2],
                             bsem[1 - buf2])
            pltpu.async_copy(xe2.at[pl.ds(nb, BLK)], xev[1 - buf2],
                             bsem[1 - buf2])
        bb = rbase + blk * BLK
        pltpu.make_async_copy(i2.at[pl.ds(bb, BLK)], ii[buf3],
                              bsem[buf2]).wait()
        pltpu.make_async_copy(j2.at[pl.ds(bb, BLK)], jj[buf2],
                              bsem[buf2]).wait()
        pltpu.make_async_copy(xe2.at[pl.ds(bb, BLK)], xev[buf2],
                              bsem[buf2]).wait()

        def pair(p, _):
            e = 2 * p
            o = e + 1
            s_wait(0, buf3, e)
            g_start(0, buf2, e)
            s_wait(1, buf3, o)
            g_start(1, buf2, o)
            g_wait(0, buf2, e)
            scale(0, buf2, e)
            s_start(0, buf3, e)
            g_wait(1, buf2, o)
            scale(1, buf2, o)
            s_start(1, buf3, o)
            return 0

        if blk == 0:
            g_start(0, buf2, 0)
            g_start(1, buf2, 1)
            g_wait(0, buf2, 0)
            scale(0, buf2, 0)
            s_start(0, buf3, 0)
            g_wait(1, buf2, 1)
            scale(1, buf2, 1)
            s_start(1, buf3, 1)
            lax.fori_loop(1, BLK // 2, pair, 0)
        else:
            lax.fori_loop(0, BLK // 2, pair, 0)

    lb3 = (NBLK - 1) % 3
    s_wait(0, lb3, BLK - 2)
    s_wait(1, lb3, BLK - 1)

    plsc.subcore_barrier()
    pltpu.sync_copy(acc_sh.at[pl.ds(row0, ROWS_PER_TILE)],
                    acc_hbm.at[c, pl.ds(row0, ROWS_PER_TILE), :])


def _sc_rows(i2, j2, xe2, h):
    mesh = plsc.VectorSubcoreMesh(core_axis_name="c", subcore_axis_name="s")
    f = pl.kernel(
        _rows_body,
        out_type=jax.ShapeDtypeStruct((NC, N_PAD, D), jnp.float32),
        mesh=mesh,
        compiler_params=pltpu.CompilerParams(needs_layout_passes=False),
        scratch_types=[
            pltpu.VMEM((BLK, CHUNK), jnp.int32),
            pltpu.VMEM((BLK, CHUNK), jnp.int32),
            pltpu.VMEM((BLK, CHUNK), jnp.int32),
            pltpu.VMEM((BLK, CHUNK), jnp.int32),
            pltpu.VMEM((BLK, CHUNK), jnp.int32),
            pltpu.VMEM((BLK, CHUNK), jnp.float32),
            pltpu.VMEM((BLK, CHUNK), jnp.float32),
            pltpu.VMEM((CHUNK, D), jnp.float32),
            pltpu.VMEM((CHUNK, D), jnp.float32),
            pltpu.VMEM((CHUNK,), jnp.float32),
            pltpu.VMEM_SHARED((N_PAD, D), jnp.float32),
            pltpu.SemaphoreType.DMA,
            pltpu.SemaphoreType.DMA,
            pltpu.SemaphoreType.DMA,
            pltpu.SemaphoreType.DMA,
            pltpu.SemaphoreType.DMA,
            pltpu.SemaphoreType.DMA,
        ],
    )
    return f(i2, j2, xe2, h)


def _fin_body(acc_ref, den_ref, out_ref):
    a = acc_ref[0] + acc_ref[1]
    d = den_ref[0] + den_ref[1] + 1e-12
    out_ref[...] = a / d


def _finalize(acc, den3):
    blk = 2000
    grid = N_NODES // blk
    return pl.pallas_call(
        _fin_body,
        grid=(grid,),
        in_specs=[
            pl.BlockSpec((NC, blk, D), lambda g: (0, g, 0)),
            pl.BlockSpec((NC, blk, 1), lambda g: (0, g, 0)),
        ],
        out_specs=pl.BlockSpec((blk, D), lambda g: (g, 0)),
        out_shape=jax.ShapeDtypeStruct((N_NODES, D), jnp.float32),
    )(acc, den3)


@jax.jit
def kernel(X, edge_index, W, att_src, att_dst):
    i = edge_index[0].astype(jnp.int32)
    j = edge_index[1].astype(jnp.int32)
    pad = E_PAD - N_EDGES
    sent = N_NODES + (jnp.arange(pad, dtype=jnp.int32) % (N_PAD - N_NODES))
    i2 = jnp.concatenate([i, sent]).reshape(E_ROWS, CHUNK)
    j2 = jnp.concatenate([j, jnp.zeros((pad,), jnp.int32)]).reshape(
        E_ROWS, CHUNK)

    i2 = i2.reshape(N_CHUNKS, NW, CHUNK).transpose(1, 0, 2).reshape(
        E_ROWS, CHUNK)
    j2 = j2.reshape(N_CHUNKS, NW, CHUNK).transpose(1, 0, 2).reshape(
        E_ROWS, CHUNK)

    h, es, ed = _project(X, W, att_src.reshape(1, D), att_dst.reshape(1, D))
    es_pad = jnp.pad(es.reshape(-1), (0, N_PAD - N_NODES))
    ed_pad = jnp.pad(ed.reshape(-1), (0, N_PAD - N_NODES))

    xe2, den = _sc_scores(i2, j2, es_pad, ed_pad)
    acc = _sc_rows(i2, j2, xe2, h)
    return _finalize(acc, den.reshape(NC, N_PAD, 1))
